# Initial kernel scaffold; baseline (speedup 1.0000x reference)
import jax
import jax.numpy as jnp
from jax.experimental import pallas as pl


def kernel(x, edge_index, W1_l, W1_r, b1, W2_l, W2_r, b2):
    raise NotImplementedError("write your pallas kernel here")


# speedup vs baseline: 4.3826x; 4.3826x over previous
import functools

import jax
import jax.numpy as jnp
from jax import lax
from jax.experimental import pallas as pl
from jax.experimental.pallas import tpu as pltpu
from jax.experimental.pallas import tpu_sc as plsc

N_NODES = 10000
N_EDGES = 320000
D_FEAT = 128
D_HID = 128
N_LABELS = 64

NC = 2
NS = 16
NW = NC * NS
K = 128
NPAD = 10240
ROWS_PER_TILE = NPAD // NS
C = -(-N_EDGES // (NW * K))
E_PAD = NW * C * K
D_EXT = D_FEAT + 16

@functools.cache
def _make_sc_aggregate(d):
  mesh = plsc.VectorSubcoreMesh(
      core_axis_name="c", subcore_axis_name="s", num_cores=NC, num_subcores=NS
  )

  @functools.partial(
      pl.kernel,
      out_type=jax.ShapeDtypeStruct((NC, NPAD, d), jnp.float32),
      mesh=mesh,
      scratch_types=[
          pltpu.VMEM((K,), jnp.int32),
          pltpu.VMEM((K,), jnp.int32),
          pltpu.VMEM((K, d), jnp.float32),
          pltpu.VMEM_SHARED((NPAD, d), jnp.float32),
          pltpu.SemaphoreType.DMA,
      ],
      compiler_params=pltpu.CompilerParams(use_tc_tiling_on_sc=False),
  )
  def sc_aggregate(src_h, dst_h, table_h, zeros_h, out_h,
                   idx_s, idx_d, rows, acc_sh, sem):
    c = lax.axis_index("c")
    s = lax.axis_index("s")
    wid = s * NC + c

    pltpu.sync_copy(zeros_h, acc_sh.at[pl.ds(s * ROWS_PER_TILE, ROWS_PER_TILE)])
    plsc.subcore_barrier()

    base = wid * (C * K)

    def chunk(i, carry):
      off = pl.multiple_of(base + i * K, 8)
      pltpu.sync_copy(src_h.at[pl.ds(off, K)], idx_s)
      pltpu.sync_copy(dst_h.at[pl.ds(off, K)], idx_d)
      pltpu.async_copy(table_h.at[idx_s], rows, sem).wait()
      pltpu.sync_copy(rows, acc_sh.at[idx_d], add=True)
      return carry

    lax.fori_loop(0, C, chunk, 0)
    plsc.subcore_barrier()

    pltpu.sync_copy(
        acc_sh.at[pl.ds(s * ROWS_PER_TILE, ROWS_PER_TILE)],
        out_h.at[c, pl.ds(s * ROWS_PER_TILE, ROWS_PER_TILE)],
    )

  return sc_aggregate


BN = 1024


def _layer1_body(parts_ref, x_ref, wl_ref, wr_ref, b_ref, h_ref, inv_ref):
  s = parts_ref[0] + parts_ref[1]
  cnt = s[:, D_FEAT:D_FEAT + 1]
  inv = 1.0 / jnp.maximum(cnt, 1.0)
  agg = s[:, :D_FEAT] * inv
  xb = x_ref[:, :D_FEAT]
  h = jnp.dot(agg, wl_ref[...], preferred_element_type=jnp.float32)
  h += jnp.dot(xb, wr_ref[...], preferred_element_type=jnp.float32)
  h += b_ref[...]
  h_ref[...] = jnp.maximum(h, 0.0)
  inv_ref[...] = jnp.broadcast_to(inv, (BN, D_HID))


def _layer2_body(parts_ref, h_ref, inv_ref, wl_ref, wr_ref, b_ref, o_ref):
  s = parts_ref[0] + parts_ref[1]
  agg = s * inv_ref[:, 0:1]
  o = jnp.dot(agg, wl_ref[...], preferred_element_type=jnp.float32)
  o += jnp.dot(h_ref[...], wr_ref[...], preferred_element_type=jnp.float32)
  o_ref[...] = o + b_ref[...]


def _tc_layer1(parts, x_ext, W1_l, W1_r, b1):
  grid = (NPAD // BN,)
  return pl.pallas_call(
      _layer1_body,
      grid=grid,
      in_specs=[
          pl.BlockSpec((NC, BN, D_EXT), lambda i: (0, i, 0)),
          pl.BlockSpec((BN, D_EXT), lambda i: (i, 0)),
          pl.BlockSpec((D_FEAT, D_HID), lambda i: (0, 0)),
          pl.BlockSpec((D_FEAT, D_HID), lambda i: (0, 0)),
          pl.BlockSpec((1, D_HID), lambda i: (0, 0)),
      ],
      out_specs=[
          pl.BlockSpec((BN, D_HID), lambda i: (i, 0)),
          pl.BlockSpec((BN, D_HID), lambda i: (i, 0)),
      ],
      out_shape=[
          jax.ShapeDtypeStruct((NPAD, D_HID), jnp.float32),
          jax.ShapeDtypeStruct((NPAD, D_HID), jnp.float32),
      ],
  )(parts, x_ext, W1_l, W1_r, b1)


def _tc_layer2(parts, h, inv_b, W2_l, W2_r, b2):
  grid = (NPAD // BN,)
  return pl.pallas_call(
      _layer2_body,
      grid=grid,
      in_specs=[
          pl.BlockSpec((NC, BN, D_HID), lambda i: (0, i, 0)),
          pl.BlockSpec((BN, D_HID), lambda i: (i, 0)),
          pl.BlockSpec((BN, D_HID), lambda i: (i, 0)),
          pl.BlockSpec((D_HID, N_LABELS), lambda i: (0, 0)),
          pl.BlockSpec((D_HID, N_LABELS), lambda i: (0, 0)),
          pl.BlockSpec((1, N_LABELS), lambda i: (0, 0)),
      ],
      out_specs=pl.BlockSpec((BN, N_LABELS), lambda i: (i, 0)),
      out_shape=jax.ShapeDtypeStruct((NPAD, N_LABELS), jnp.float32),
  )(parts, h, inv_b, W2_l, W2_r, b2)


def kernel(x, edge_index, W1_l, W1_r, b1, W2_l, W2_r, b2):
  src = edge_index[0].astype(jnp.int32)
  dst = edge_index[1].astype(jnp.int32)
  pad = jnp.full((E_PAD - N_EDGES,), N_NODES, dtype=jnp.int32)
  src_p = jnp.concatenate([src, pad])
  dst_p = jnp.concatenate([dst, pad])

  x_ext = jnp.zeros((NPAD, D_EXT), jnp.float32)
  x_ext = x_ext.at[:N_NODES, :D_FEAT].set(x)
  x_ext = x_ext.at[:N_NODES, D_FEAT].set(1.0)

  zeros_ext = jnp.zeros((ROWS_PER_TILE, D_EXT), jnp.float32)
  zeros_hid = jnp.zeros((ROWS_PER_TILE, D_HID), jnp.float32)

  parts1 = _make_sc_aggregate(D_EXT)(src_p, dst_p, x_ext, zeros_ext)
  h, inv_b = _tc_layer1(parts1, x_ext, W1_l, W1_r, b1.reshape(1, D_HID))
  parts2 = _make_sc_aggregate(D_HID)(src_p, dst_p, h, zeros_hid)
  out = _tc_layer2(parts2, h, inv_b, W2_l, W2_r, b2.reshape(1, N_LABELS))
  return out[:N_NODES]
